# baseline (speedup 1.0000x reference)
import jax
import jax.numpy as jnp
from jax import lax
from jax.experimental import pallas as pl

_A = 5
_H = 26
_W = 26
_T = 30
_HW = _H * _W
_C = _A * _HW
_POS_THRESH = 0.6


def _body(xy_ref, wh_ref, obj_ref, truth_ref, bias_ref, brow_ref,
          xyo_ref, who_ref, g_ref, tobj_ref, tnoobj_ref, tlabel_ref):
    f32 = jnp.float32
    tx = truth_ref[0, :, 0:1]
    ty = truth_ref[0, :, 1:2]
    tw = truth_ref[0, :, 2:3]
    th = truth_ref[0, :, 3:4]
    tcls = truth_ref[0, :, 4:5]
    valid = tw > 1e-6

    twc = tw * _W
    thc = th * _H
    ci = jnp.clip((tx * _W).astype(jnp.int32), 0, _W - 1)
    cj = jnp.clip((ty * _H).astype(jnp.int32), 0, _H - 1)
    tgt_x = tx * _W - ci.astype(f32)
    tgt_y = ty * _H - cj.astype(f32)
    wgt = 2.0 - tw * th

    best_r = jnp.full_like(tx, -1.0)
    ba = jnp.zeros_like(ci)
    bw_sel = jnp.zeros_like(tx)
    bh_sel = jnp.zeros_like(tx)
    for a in range(_A):
        bw_a = bias_ref[0:1, 2 * a:2 * a + 1]
        bh_a = bias_ref[0:1, 2 * a + 1:2 * a + 2]
        inter = jnp.minimum(twc, bw_a) * jnp.minimum(thc, bh_a)
        union = twc * thc + bw_a * bh_a - inter
        r = inter / jnp.maximum(union, 1e-12)
        upd = r > best_r
        best_r = jnp.where(upd, r, best_r)
        ba = jnp.where(upd, a, ba)
        bw_sel = jnp.where(upd, bw_a, bw_sel)
        bh_sel = jnp.where(upd, bh_a, bh_sel)
    tgt_w = jnp.log(jnp.maximum(twc, 1e-12) / bw_sel)
    tgt_h = jnp.log(jnp.maximum(thc, 1e-12) / bh_sel)

    m = jnp.where(valid, ba * _HW + cj * _W + ci, -1)
    half_tw = tw * 0.5
    half_th = th * 0.5
    tl = jnp.where(valid, tx - half_tw, 1e30)
    tr = jnp.where(valid, tx + half_tw, -1e30)
    tt = ty - half_th
    tb = ty + half_th
    t_area = tw * th
    c1 = t_area * (_POS_THRESH / (1.0 + _POS_THRESH))

    tcol1 = lax.broadcasted_iota(jnp.int32, (_T, 1), 0) + 1
    idx_row = lax.broadcasted_iota(jnp.int32, (1, _C), 1)
    gx = (idx_row % _W).astype(f32)
    gy = ((idx_row // _W) % _H).astype(f32)

    xy0 = xy_ref[0, 0:1, :]
    xy1 = xy_ref[0, 1:2, :]
    wh0 = wh_ref[0, 0:1, :]
    wh1 = wh_ref[0, 1:2, :]
    obj_r = obj_ref[0, 0:1, :]
    bw_row = brow_ref[0, 0:1, :]
    bh_row = brow_ref[0, 1:2, :]

    px = (gx + xy0) * (1.0 / _W)
    py = (gy + xy1) * (1.0 / _H)
    pw = jnp.exp(wh0) * bw_row
    ph = jnp.exp(wh1) * bh_row
    p_l = px - pw * 0.5
    p_r = px + pw * 0.5
    p_t = py - ph * 0.5
    p_b = py + ph * 0.5
    p_area = pw * ph
    c0 = p_area * (_POS_THRESH / (1.0 + _POS_THRESH))

    l = jnp.maximum(p_l, tl)
    r = jnp.minimum(p_r, tr)
    t = jnp.maximum(p_t, tt)
    b = jnp.minimum(p_b, tb)
    inter = jnp.maximum(r - l, 0.0) * jnp.maximum(b - t, 0.0)
    over = inter > (c0 + c1)
    any_over = jnp.max(over.astype(f32), axis=0, keepdims=True) > 0.0

    match_i = jnp.where(m == idx_row, tcol1, 0)
    selid = jnp.max(match_i, axis=0, keepdims=True)
    assigned = selid > 0
    wf = ((match_i == selid) & assigned).astype(f32)

    union = (p_area + t_area) - inter
    inter_w = jnp.sum(wf * inter, axis=0, keepdims=True)
    union_w = jnp.sum(wf * union, axis=0, keepdims=True)

    vals = jnp.concatenate([tgt_x, tgt_y, tgt_w, tgt_h, wgt, tcls], axis=1)
    picked = lax.dot_general(vals, wf, (((0,), (0,)), ((), ())),
                             preferred_element_type=f32,
                             precision=lax.Precision.HIGHEST)

    xyo_ref[0, 0:1, :] = jnp.where(assigned, picked[0:1], xy0)
    xyo_ref[0, 1:2, :] = jnp.where(assigned, picked[1:2], xy1)
    who_ref[0, 0:1, :] = jnp.where(assigned, picked[2:3], wh0)
    who_ref[0, 1:2, :] = jnp.where(assigned, picked[3:4], wh1)
    g_ref[0, 0:1, :] = jnp.where(assigned, picked[4:5], 0.0)
    iou_w = inter_w / jnp.maximum(union_w, 1e-12)
    tobj_ref[0, 0:1, :] = jnp.where(assigned, iou_w, obj_r)
    tnoobj_ref[0, 0:1, :] = jnp.where(assigned | any_over, obj_r, 0.0)
    tlabel_ref[0, 0:1, :] = jnp.where(assigned, picked[5:6], -1.0)


def kernel(xy, wh, obj, truth, biases):
    B = xy.shape[0]
    f32 = jnp.float32
    xy_t = xy.reshape(B, _A, 2, _HW).transpose(0, 2, 1, 3).reshape(B, 2, _C)
    wh_t = wh.reshape(B, _A, 2, _HW).transpose(0, 2, 1, 3).reshape(B, 2, _C)
    obj_r = obj.reshape(B, 1, _C)
    bias_r = biases.reshape(1, 2 * _A)
    bi = biases.reshape(_A, 2)
    brow = jnp.stack([jnp.repeat(bi[:, 0] * (1.0 / _W), _HW),
                      jnp.repeat(bi[:, 1] * (1.0 / _H), _HW)], axis=0)
    brow = brow.reshape(1, 2, _C)

    out_shapes = (
        jax.ShapeDtypeStruct((B, 2, _C), f32),
        jax.ShapeDtypeStruct((B, 2, _C), f32),
        jax.ShapeDtypeStruct((B, 1, _C), f32),
        jax.ShapeDtypeStruct((B, 1, _C), f32),
        jax.ShapeDtypeStruct((B, 1, _C), f32),
        jax.ShapeDtypeStruct((B, 1, _C), f32),
    )
    in_specs = [
        pl.BlockSpec((1, 2, _C), lambda b: (b, 0, 0)),
        pl.BlockSpec((1, 2, _C), lambda b: (b, 0, 0)),
        pl.BlockSpec((1, 1, _C), lambda b: (b, 0, 0)),
        pl.BlockSpec((1, _T, 5), lambda b: (b, 0, 0)),
        pl.BlockSpec((1, 2 * _A), lambda b: (0, 0)),
        pl.BlockSpec((1, 2, _C), lambda b: (0, 0, 0)),
    ]
    out_specs = (
        pl.BlockSpec((1, 2, _C), lambda b: (b, 0, 0)),
        pl.BlockSpec((1, 2, _C), lambda b: (b, 0, 0)),
        pl.BlockSpec((1, 1, _C), lambda b: (b, 0, 0)),
        pl.BlockSpec((1, 1, _C), lambda b: (b, 0, 0)),
        pl.BlockSpec((1, 1, _C), lambda b: (b, 0, 0)),
        pl.BlockSpec((1, 1, _C), lambda b: (b, 0, 0)),
    )
    xyo, who, g, tobj, tnoobj, tlabel = pl.pallas_call(
        _body,
        grid=(B,),
        in_specs=in_specs,
        out_specs=out_specs,
        out_shape=out_shapes,
    )(xy_t, wh_t, obj_r, truth, bias_r, brow)

    t_xy = xyo.reshape(B, 2, _A, _HW).transpose(0, 2, 1, 3).reshape(
        B, 2 * _A, _H, _W)
    t_wh = who.reshape(B, 2, _A, _HW).transpose(0, 2, 1, 3).reshape(
        B, 2 * _A, _H, _W)
    g4 = g.reshape(B, 1, _A, _HW)
    t_w = jnp.broadcast_to(g4, (B, 2, _A, _HW)).transpose(0, 2, 1, 3).reshape(
        B, 2 * _A, _H, _W)
    return (
        t_xy,
        t_wh,
        t_w,
        tobj.reshape(B, _A, _H, _W),
        tnoobj.reshape(B, _A, _H, _W),
        tlabel.reshape(B, _A, _H, _W),
    )
